# baseline (speedup 1.0000x reference)
import functools
import math

import jax
import jax.numpy as jnp
from jax import lax
from jax.experimental import pallas as pl
from jax.experimental.pallas import tpu as pltpu
from jax.experimental.pallas import tpu_sc as plsc

N = 10000
D = 128
E = 320000
S = 100
ES = 1600
H = 4
DH = D // H
FF = 512
NC2 = 2

NCORES = 2
NSUB = 16
NW = NCORES * NSUB
CH = 128
GRP = 8
GROUPS = 10
CPW = GRP * GROUPS
EPAD = NW * CPW * CH
SUB = 64
SGRP = 8
NBODY = EPAD // (NW * SUB * SGRP)
NBUF = 4
NP = 10240
NZCH = NP // CH
ZPT = NZCH // NSUB
LBN = 2560

BN = 1000
GRID_N = N // BN

_EPS = 1e-6


def _lnorm(z, g, b):
    mu = jnp.mean(z, axis=-1, keepdims=True)
    var = jnp.mean((z - mu) ** 2, axis=-1, keepdims=True)
    return (z - mu) / jnp.sqrt(var + _EPS) * g + b


def _dot(a, b):
    return jnp.dot(a, b, preferred_element_type=jnp.float32)


def _ln_body(x_ref, g_ref, b_ref, o_ref):
    o_ref[...] = _lnorm(x_ref[...], g_ref[...], b_ref[...])


def _stage_ln(x, g, b):
    return pl.pallas_call(
        _ln_body,
        grid=(NP // LBN,),
        in_specs=[
            pl.BlockSpec((LBN, D), lambda i: (i, 0)),
            pl.BlockSpec((1, D), lambda i: (0, 0)),
            pl.BlockSpec((1, D), lambda i: (0, 0)),
        ],
        out_specs=pl.BlockSpec((LBN, D), lambda i: (i, 0)),
        out_shape=jax.ShapeDtypeStruct((NP, D), jnp.float32),
    )(x, g, b)


def _sc_body(h_hbm, src_hbm, dst_hbm, zr_hbm,
             out_hbm, cnt_hbm,
             src_v, dst_v, r0, r1, r2, r3,
             cnt_v, acc_sh, g0, g1, g2, g3, s0, s1, s2, s3):
    cid = lax.axis_index("c")
    sid = lax.axis_index("s")
    wid = cid * NSUB + sid
    rows = (r0, r1, r2, r3)
    gsem = (g0, g1, g2, g3)
    ssem = (s0, s1, s2, s3)

    def zcnt(i, carry):
        cnt_v[pl.ds(pl.multiple_of(i * 16, 8), 16)] = jnp.zeros(
            (16,), jnp.float32)
        return carry

    lax.fori_loop(0, NP // 16, zcnt, 0)

    pltpu.sync_copy(zr_hbm, r0)
    for k in range(NP // NSUB // SUB):
        start = pl.multiple_of(sid * (NP // NSUB) + k * SUB, 8)
        pltpu.sync_copy(r0, acc_sh.at[pl.ds(start, SUB)])

    plsc.subcore_barrier()

    ones16 = jnp.ones((16,), jnp.float32)

    def body(g, carry):
        base = pl.multiple_of(g * SGRP, 8)
        pltpu.sync_copy(src_hbm.at[wid, pl.ds(base, SGRP)], src_v)
        pltpu.sync_copy(dst_hbm.at[wid, pl.ds(base, SGRP)], dst_v)
        for u in range(NBUF - 1):
            pltpu.async_copy(h_hbm.at[src_v.at[u]], rows[u], gsem[u])
        for u in range(SGRP):
            b = u % NBUF
            pltpu.make_async_copy(h_hbm.at[src_v.at[u]], rows[b],
                                  gsem[b]).wait()
            for k in range(SUB // 16):
                idx = dst_v[u, pl.ds(k * 16, 16)]
                plsc.addupdate_scatter(cnt_v, [idx], ones16)
            un = u + NBUF - 1
            if un < SGRP:
                bn = un % NBUF
                if u >= 1:
                    pltpu.make_async_copy(
                        rows[bn], acc_sh.at[dst_v.at[u - 1]],
                        ssem[bn]).wait()
                pltpu.async_copy(h_hbm.at[src_v.at[un]], rows[bn], gsem[bn])
            pltpu.async_copy(rows[b], acc_sh.at[dst_v.at[u]], ssem[b],
                             add=True)
        for u in range(SGRP - NBUF, SGRP):
            b = u % NBUF
            pltpu.make_async_copy(rows[b], acc_sh.at[dst_v.at[u]],
                                  ssem[b]).wait()
        return carry

    lax.fori_loop(0, NBODY, body, 0)
    plsc.subcore_barrier()

    for k in range(NP // NSUB // SUB):
        start = pl.multiple_of(sid * (NP // NSUB) + k * SUB, 8)
        pltpu.sync_copy(acc_sh.at[pl.ds(start, SUB)], r0)
        pltpu.sync_copy(r0, out_hbm.at[cid, pl.ds(start, SUB)])

    pltpu.sync_copy(cnt_v, cnt_hbm.at[wid])


def _stage_sc(h, src3d, dst3d):
    zrows = jnp.zeros((SUB, D), jnp.float32)
    mesh = plsc.VectorSubcoreMesh(
        core_axis_name="c", subcore_axis_name="s",
        num_cores=NCORES, num_subcores=NSUB)
    f = pl.kernel(
        _sc_body,
        compiler_params=pltpu.CompilerParams(needs_layout_passes=False),
        out_type=(
            pltpu.HBM((NCORES, NP, D), jnp.float32),
            pltpu.HBM((NW, NP), jnp.float32),
        ),
        mesh=mesh,
        scratch_types=[
            pltpu.VMEM((SGRP, SUB), jnp.int32),
            pltpu.VMEM((SGRP, SUB), jnp.int32),
            pltpu.VMEM((SUB, D), jnp.float32),
            pltpu.VMEM((SUB, D), jnp.float32),
            pltpu.VMEM((SUB, D), jnp.float32),
            pltpu.VMEM((SUB, D), jnp.float32),
            pltpu.VMEM((NP,), jnp.float32),
            pltpu.VMEM_SHARED((NP, D), jnp.float32),
            pltpu.SemaphoreType.DMA,
            pltpu.SemaphoreType.DMA,
            pltpu.SemaphoreType.DMA,
            pltpu.SemaphoreType.DMA,
            pltpu.SemaphoreType.DMA,
            pltpu.SemaphoreType.DMA,
            pltpu.SemaphoreType.DMA,
            pltpu.SemaphoreType.DMA,
        ],
    )
    return f(h, src3d, dst3d, zrows)


def _cnt_body(c_ref, one_ref, o_ref):
    o_ref[...] = lax.dot_general(c_ref[...], one_ref[...],
                                 (((0,), (0,)), ((), ())),
                                 preferred_element_type=jnp.float32)


def _stage_cnt(cnt):
    ones_w = jnp.ones((NW, 1), jnp.float32)
    return pl.pallas_call(
        _cnt_body,
        grid=(NP // LBN,),
        in_specs=[
            pl.BlockSpec((NW, LBN), lambda i: (0, i)),
            pl.BlockSpec((NW, 1), lambda i: (0, 0)),
        ],
        out_specs=pl.BlockSpec((LBN, 1), lambda i: (i, 0)),
        out_shape=jax.ShapeDtypeStruct((NP, 1), jnp.float32),
    )(cnt, ones_w)


def _mid_body(h_ref, p_ref, c_ref, seg_ref, wl_ref, wr_ref, b_ref,
              g_ref, be_ref, pooled_ref, pcnt_ref):
    i = pl.program_id(0)
    h = h_ref[...]
    p = p_ref[...]
    agg = p[0] + p[1]
    cnt = jnp.maximum(c_ref[...], 1.0)
    aggm = agg / cnt
    y = _dot(h, wl_ref[...]) + _dot(aggm, wr_ref[...]) + b_ref[...]
    y = jnp.maximum(y, 0.0)
    yn = _lnorm(y, g_ref[...], be_ref[...])

    segrow = seg_ref[0]
    rows = lax.broadcasted_iota(jnp.int32, (D, BN), 0)
    oh = (rows == segrow).astype(jnp.float32)
    pooledb = _dot(oh, yn)
    pcntb = jnp.sum(oh, axis=1, keepdims=True)

    @pl.when(i == 0)
    def _():
        pooled_ref[...] = jnp.zeros_like(pooled_ref)
        pcnt_ref[...] = jnp.zeros_like(pcnt_ref)

    pooled_ref[...] += pooledb
    pcnt_ref[...] += pcntb


def _stage_mid(h, part, cnt, seg3d, W1l, W1r, b1, g, be):
    return pl.pallas_call(
        _mid_body,
        grid=(GRID_N,),
        in_specs=[
            pl.BlockSpec((BN, D), lambda i: (i, 0)),
            pl.BlockSpec((NCORES, BN, D), lambda i: (0, i, 0)),
            pl.BlockSpec((BN, 1), lambda i: (i, 0)),
            pl.BlockSpec((1, 1, BN), lambda i: (i, 0, 0)),
            pl.BlockSpec((D, D), lambda i: (0, 0)),
            pl.BlockSpec((D, D), lambda i: (0, 0)),
            pl.BlockSpec((1, D), lambda i: (0, 0)),
            pl.BlockSpec((1, D), lambda i: (0, 0)),
            pl.BlockSpec((1, D), lambda i: (0, 0)),
        ],
        out_specs=[
            pl.BlockSpec((D, D), lambda i: (0, 0)),
            pl.BlockSpec((D, 1), lambda i: (0, 0)),
        ],
        out_shape=[
            jax.ShapeDtypeStruct((D, D), jnp.float32),
            jax.ShapeDtypeStruct((D, 1), jnp.float32),
        ],
    )(h, part, cnt, seg3d, W1l, W1r, b1, g, be)


def _head_body(pooled_ref, pcnt_ref, es_ref,
               g2i_ref, b2i_ref, w2l_ref, w2r_ref, b2_ref, g2o_ref, b2o_ref,
               pew_ref, peb_ref, cls_ref, pos_ref, preg_ref, preb_ref,
               lag_ref, lab_ref, wqkv_ref, bqkv_ref, wo_ref, bo_ref,
               lmg_ref, lmb_ref, wf1_ref, bf1_ref, wf2_ref, bf2_ref,
               lfg_ref, lfb_ref, hw_ref, hb_ref, out_ref):
    cnt = jnp.maximum(pcnt_ref[...], 1.0)
    fea = pooled_ref[...] / cnt
    g = _lnorm(fea, g2i_ref[...], b2i_ref[...])

    es = es_ref[...]
    rows = lax.broadcasted_iota(jnp.int32, (D, ES), 0)
    od = (rows == es[1:2, :]).astype(jnp.float32)
    osrc = (rows == es[0:1, :]).astype(jnp.float32)
    adj = lax.dot_general(od, osrc, (((1,), (1,)), ((), ())),
                          preferred_element_type=jnp.float32)
    deg = jnp.sum(od, axis=1, keepdims=True)
    agg2 = _dot(adj, g) / jnp.maximum(deg, 1.0)
    g2 = _dot(g, w2l_ref[...]) + _dot(agg2, w2r_ref[...]) + b2_ref[...]
    g2 = jnp.maximum(g2, 0.0)
    gg = _lnorm(g2, g2o_ref[...], b2o_ref[...])

    proj = _dot(gg, pew_ref[...]) + peb_ref[...]
    t = jnp.concatenate([cls_ref[...], proj[0:S, :]], axis=0)
    t = t + pos_ref[...]
    t = _lnorm(t, preg_ref[...], preb_ref[...])

    a = _lnorm(t, lag_ref[...], lab_ref[...])
    qkv = _dot(a, wqkv_ref[...]) + bqkv_ref[...]
    heads = []
    scale = 1.0 / math.sqrt(DH)
    for hh in range(H):
        qh = qkv[:, hh * DH:(hh + 1) * DH]
        kh = qkv[:, D + hh * DH:D + (hh + 1) * DH]
        vh = qkv[:, 2 * D + hh * DH:2 * D + (hh + 1) * DH]
        lg = lax.dot_general(qh, kh, (((1,), (1,)), ((), ())),
                             preferred_element_type=jnp.float32) * scale
        m = jnp.max(lg, axis=-1, keepdims=True)
        e = jnp.exp(lg - m)
        att = e / jnp.sum(e, axis=-1, keepdims=True)
        heads.append(_dot(att, vh))
    o = jnp.concatenate(heads, axis=1)
    t = t + _dot(o, wo_ref[...]) + bo_ref[...]
    mm = _lnorm(t, lmg_ref[...], lmb_ref[...])
    ff = _dot(jax.nn.gelu(_dot(mm, wf1_ref[...]) + bf1_ref[...]),
              wf2_ref[...]) + bf2_ref[...]
    t = t + ff
    t = _lnorm(t, lfg_ref[...], lfb_ref[...])
    out_ref[...] = _dot(t[0:1, :], hw_ref[...]) + hb_ref[...]


def _stage_head(pooled, pcnt, es, *ws):
    return pl.pallas_call(
        _head_body,
        out_shape=jax.ShapeDtypeStruct((1, NC2), jnp.float32),
    )(pooled, pcnt, es, *ws)


def kernel(x, edge_patch, superpixel_attri, edge_superpixel, ln_in1_g,
           ln_in1_b, W1l, W1r, b1, ln_out1_g, ln_out1_b, ln_in2_g, ln_in2_b,
           W2l, W2r, b2, ln_out2_g, ln_out2_b, pe_W, pe_b, cls, pos, pre_g,
           pre_b, lnA_g, lnA_b, Wqkv, bqkv, Wo, bo, lnM_g, lnM_b, Wf1, bf1,
           Wf2, bf2, lnF_g, lnF_b, head_W, head_b):
    r = lambda v: v.reshape(1, -1)
    pad = N + jnp.arange(EPAD - E, dtype=jnp.int32) % (NP - N)
    src3d = jnp.concatenate([edge_patch[0], pad]).reshape(
        NW, NBODY * SGRP, SUB)
    dst3d = jnp.concatenate([edge_patch[1], pad]).reshape(
        NW, NBODY * SGRP, SUB)
    seg3d = superpixel_attri.reshape(GRID_N, 1, BN)
    x_pad = jnp.pad(x, ((0, NP - N), (0, 0)))

    h = _stage_ln(x_pad, r(ln_in1_g), r(ln_in1_b))
    part, cnt = _stage_sc(h, src3d, dst3d)
    cnt_col = _stage_cnt(cnt)
    pooled, pcnt = _stage_mid(h, part, cnt_col, seg3d, W1l, W1r, r(b1),
                              r(ln_out1_g), r(ln_out1_b))
    out = _stage_head(
        pooled, pcnt, edge_superpixel,
        r(ln_in2_g), r(ln_in2_b), W2l, W2r, r(b2), r(ln_out2_g), r(ln_out2_b),
        pe_W, r(pe_b), cls.reshape(1, D), pos.reshape(S + 1, D),
        r(pre_g), r(pre_b), r(lnA_g), r(lnA_b), Wqkv, r(bqkv), Wo, r(bo),
        r(lnM_g), r(lnM_b), Wf1, r(bf1), Wf2, r(bf2), r(lnF_g), r(lnF_b),
        head_W, r(head_b))
    return out


# speedup vs baseline: 1.1045x; 1.1045x over previous
import functools
import math

import jax
import jax.numpy as jnp
from jax import lax
from jax.experimental import pallas as pl
from jax.experimental.pallas import tpu as pltpu
from jax.experimental.pallas import tpu_sc as plsc

N = 10000
D = 128
E = 320000
S = 100
ES = 1600
H = 4
DH = D // H
FF = 512
NC2 = 2

NCORES = 2
NSUB = 16
NW = NCORES * NSUB
CH = 128
GRP = 8
GROUPS = 10
CPW = GRP * GROUPS
EPAD = NW * CPW * CH
SUB = 64
SGRP = 8
NBODY = EPAD // (NW * SUB * SGRP)
NBUF = 4
NP = 10240
NZCH = NP // CH
ZPT = NZCH // NSUB
LBN = 2560

BN = 1000
GRID_N = N // BN

_EPS = 1e-6


def _lnorm(z, g, b):
    mu = jnp.mean(z, axis=-1, keepdims=True)
    var = jnp.mean((z - mu) ** 2, axis=-1, keepdims=True)
    return (z - mu) / jnp.sqrt(var + _EPS) * g + b


def _dot(a, b):
    return jnp.dot(a, b, preferred_element_type=jnp.float32)


def _ln_body(x_ref, g_ref, b_ref, o_ref):
    o_ref[...] = _lnorm(x_ref[...], g_ref[...], b_ref[...])


def _stage_ln(x, g, b):
    return pl.pallas_call(
        _ln_body,
        grid=(NP // LBN,),
        in_specs=[
            pl.BlockSpec((LBN, D), lambda i: (i, 0)),
            pl.BlockSpec((1, D), lambda i: (0, 0)),
            pl.BlockSpec((1, D), lambda i: (0, 0)),
        ],
        out_specs=pl.BlockSpec((LBN, D), lambda i: (i, 0)),
        out_shape=jax.ShapeDtypeStruct((NP, D), jnp.float32),
    )(x, g, b)


def _sc_body(h_hbm, src_hbm, dst_hbm, zr_hbm,
             out_hbm, cnt_hbm,
             src_v, dst_v, r0, r1, r2, r3,
             cnt_v, acc_sh, g0, g1, g2, g3, s0, s1, s2, s3):
    cid = lax.axis_index("c")
    sid = lax.axis_index("s")
    wid = cid * NSUB + sid
    rows = (r0, r1, r2, r3)
    gsem = (g0, g1, g2, g3)
    ssem = (s0, s1, s2, s3)

    def zcnt(i, carry):
        cnt_v[pl.ds(pl.multiple_of(i * 16, 8), 16)] = jnp.zeros(
            (16,), jnp.float32)
        return carry

    lax.fori_loop(0, NP // 16, zcnt, 0)

    pltpu.sync_copy(zr_hbm, r0)
    for k in range(NP // NSUB // SUB):
        start = pl.multiple_of(sid * (NP // NSUB) + k * SUB, 8)
        pltpu.sync_copy(r0, acc_sh.at[pl.ds(start, SUB)])

    plsc.subcore_barrier()

    ones16 = jnp.ones((16,), jnp.float32)

    def body(g, carry):
        base = pl.multiple_of(g * 2 * SGRP, 8)
        pltpu.sync_copy(src_hbm.at[wid, pl.ds(base, 2 * SGRP)], src_v)
        pltpu.sync_copy(dst_hbm.at[wid, pl.ds(base, 2 * SGRP)], dst_v)
        for u in range(NBUF - 1):
            pltpu.async_copy(h_hbm.at[src_v.at[u]], rows[u], gsem[u])
        for u in range(2 * SGRP):
            b = u % NBUF
            pltpu.make_async_copy(h_hbm.at[src_v.at[u]], rows[b],
                                  gsem[b]).wait()
            for k in range(SUB // 16):
                idx = dst_v[u, pl.ds(k * 16, 16)]
                plsc.addupdate_scatter(cnt_v, [idx], ones16)
            un = u + NBUF - 1
            if un < 2 * SGRP:
                pltpu.async_copy(h_hbm.at[src_v.at[un]], rows[un % NBUF],
                                 gsem[un % NBUF])
            pltpu.sync_copy(rows[b], acc_sh.at[dst_v.at[u]], add=True)
        return carry

    lax.fori_loop(0, NBODY // 2, body, 0)
    plsc.subcore_barrier()

    for k in range(NP // NSUB // SUB):
        start = pl.multiple_of(sid * (NP // NSUB) + k * SUB, 8)
        pltpu.sync_copy(acc_sh.at[pl.ds(start, SUB)], r0)
        pltpu.sync_copy(r0, out_hbm.at[cid, pl.ds(start, SUB)])

    pltpu.sync_copy(cnt_v, cnt_hbm.at[wid])


def _stage_sc(h, src3d, dst3d):
    zrows = jnp.zeros((SUB, D), jnp.float32)
    mesh = plsc.VectorSubcoreMesh(
        core_axis_name="c", subcore_axis_name="s",
        num_cores=NCORES, num_subcores=NSUB)
    f = pl.kernel(
        _sc_body,
        compiler_params=pltpu.CompilerParams(needs_layout_passes=False),
        out_type=(
            pltpu.HBM((NCORES, NP, D), jnp.float32),
            pltpu.HBM((NW, NP), jnp.float32),
        ),
        mesh=mesh,
        scratch_types=[
            pltpu.VMEM((2 * SGRP, SUB), jnp.int32),
            pltpu.VMEM((2 * SGRP, SUB), jnp.int32),
            pltpu.VMEM((SUB, D), jnp.float32),
            pltpu.VMEM((SUB, D), jnp.float32),
            pltpu.VMEM((SUB, D), jnp.float32),
            pltpu.VMEM((SUB, D), jnp.float32),
            pltpu.VMEM((NP,), jnp.float32),
            pltpu.VMEM_SHARED((NP, D), jnp.float32),
            pltpu.SemaphoreType.DMA,
            pltpu.SemaphoreType.DMA,
            pltpu.SemaphoreType.DMA,
            pltpu.SemaphoreType.DMA,
            pltpu.SemaphoreType.DMA,
            pltpu.SemaphoreType.DMA,
            pltpu.SemaphoreType.DMA,
            pltpu.SemaphoreType.DMA,
        ],
    )
    return f(h, src3d, dst3d, zrows)


def _cnt_body(c_ref, one_ref, o_ref):
    o_ref[...] = lax.dot_general(c_ref[...], one_ref[...],
                                 (((0,), (0,)), ((), ())),
                                 preferred_element_type=jnp.float32)


def _stage_cnt(cnt):
    ones_w = jnp.ones((NW, 1), jnp.float32)
    return pl.pallas_call(
        _cnt_body,
        grid=(NP // LBN,),
        in_specs=[
            pl.BlockSpec((NW, LBN), lambda i: (0, i)),
            pl.BlockSpec((NW, 1), lambda i: (0, 0)),
        ],
        out_specs=pl.BlockSpec((LBN, 1), lambda i: (i, 0)),
        out_shape=jax.ShapeDtypeStruct((NP, 1), jnp.float32),
    )(cnt, ones_w)


def _mid_body(h_ref, p_ref, c_ref, seg_ref, wl_ref, wr_ref, b_ref,
              g_ref, be_ref, pooled_ref, pcnt_ref):
    i = pl.program_id(0)
    h = h_ref[...]
    p = p_ref[...]
    agg = p[0] + p[1]
    cnt = jnp.maximum(c_ref[...], 1.0)
    aggm = agg / cnt
    y = _dot(h, wl_ref[...]) + _dot(aggm, wr_ref[...]) + b_ref[...]
    y = jnp.maximum(y, 0.0)
    yn = _lnorm(y, g_ref[...], be_ref[...])

    segrow = seg_ref[0]
    rows = lax.broadcasted_iota(jnp.int32, (D, BN), 0)
    oh = (rows == segrow).astype(jnp.float32)
    pooledb = _dot(oh, yn)
    pcntb = jnp.sum(oh, axis=1, keepdims=True)

    @pl.when(i == 0)
    def _():
        pooled_ref[...] = jnp.zeros_like(pooled_ref)
        pcnt_ref[...] = jnp.zeros_like(pcnt_ref)

    pooled_ref[...] += pooledb
    pcnt_ref[...] += pcntb


def _stage_mid(h, part, cnt, seg3d, W1l, W1r, b1, g, be):
    return pl.pallas_call(
        _mid_body,
        grid=(GRID_N,),
        in_specs=[
            pl.BlockSpec((BN, D), lambda i: (i, 0)),
            pl.BlockSpec((NCORES, BN, D), lambda i: (0, i, 0)),
            pl.BlockSpec((BN, 1), lambda i: (i, 0)),
            pl.BlockSpec((1, 1, BN), lambda i: (i, 0, 0)),
            pl.BlockSpec((D, D), lambda i: (0, 0)),
            pl.BlockSpec((D, D), lambda i: (0, 0)),
            pl.BlockSpec((1, D), lambda i: (0, 0)),
            pl.BlockSpec((1, D), lambda i: (0, 0)),
            pl.BlockSpec((1, D), lambda i: (0, 0)),
        ],
        out_specs=[
            pl.BlockSpec((D, D), lambda i: (0, 0)),
            pl.BlockSpec((D, 1), lambda i: (0, 0)),
        ],
        out_shape=[
            jax.ShapeDtypeStruct((D, D), jnp.float32),
            jax.ShapeDtypeStruct((D, 1), jnp.float32),
        ],
    )(h, part, cnt, seg3d, W1l, W1r, b1, g, be)


def _head_body(pooled_ref, pcnt_ref, es_ref,
               g2i_ref, b2i_ref, w2l_ref, w2r_ref, b2_ref, g2o_ref, b2o_ref,
               pew_ref, peb_ref, cls_ref, pos_ref, preg_ref, preb_ref,
               lag_ref, lab_ref, wqkv_ref, bqkv_ref, wo_ref, bo_ref,
               lmg_ref, lmb_ref, wf1_ref, bf1_ref, wf2_ref, bf2_ref,
               lfg_ref, lfb_ref, hw_ref, hb_ref, out_ref):
    cnt = jnp.maximum(pcnt_ref[...], 1.0)
    fea = pooled_ref[...] / cnt
    g = _lnorm(fea, g2i_ref[...], b2i_ref[...])

    es = es_ref[...]
    rows = lax.broadcasted_iota(jnp.int32, (D, ES), 0)
    od = (rows == es[1:2, :]).astype(jnp.float32)
    osrc = (rows == es[0:1, :]).astype(jnp.float32)
    adj = lax.dot_general(od, osrc, (((1,), (1,)), ((), ())),
                          preferred_element_type=jnp.float32)
    deg = jnp.sum(od, axis=1, keepdims=True)
    agg2 = _dot(adj, g) / jnp.maximum(deg, 1.0)
    g2 = _dot(g, w2l_ref[...]) + _dot(agg2, w2r_ref[...]) + b2_ref[...]
    g2 = jnp.maximum(g2, 0.0)
    gg = _lnorm(g2, g2o_ref[...], b2o_ref[...])

    proj = _dot(gg, pew_ref[...]) + peb_ref[...]
    t = jnp.concatenate([cls_ref[...], proj[0:S, :]], axis=0)
    t = t + pos_ref[...]
    t = _lnorm(t, preg_ref[...], preb_ref[...])

    a = _lnorm(t, lag_ref[...], lab_ref[...])
    qkv = _dot(a, wqkv_ref[...]) + bqkv_ref[...]
    heads = []
    scale = 1.0 / math.sqrt(DH)
    for hh in range(H):
        qh = qkv[:, hh * DH:(hh + 1) * DH]
        kh = qkv[:, D + hh * DH:D + (hh + 1) * DH]
        vh = qkv[:, 2 * D + hh * DH:2 * D + (hh + 1) * DH]
        lg = lax.dot_general(qh, kh, (((1,), (1,)), ((), ())),
                             preferred_element_type=jnp.float32) * scale
        m = jnp.max(lg, axis=-1, keepdims=True)
        e = jnp.exp(lg - m)
        att = e / jnp.sum(e, axis=-1, keepdims=True)
        heads.append(_dot(att, vh))
    o = jnp.concatenate(heads, axis=1)
    t = t + _dot(o, wo_ref[...]) + bo_ref[...]
    mm = _lnorm(t, lmg_ref[...], lmb_ref[...])
    ff = _dot(jax.nn.gelu(_dot(mm, wf1_ref[...]) + bf1_ref[...]),
              wf2_ref[...]) + bf2_ref[...]
    t = t + ff
    t = _lnorm(t, lfg_ref[...], lfb_ref[...])
    out_ref[...] = _dot(t[0:1, :], hw_ref[...]) + hb_ref[...]


def _stage_head(pooled, pcnt, es, *ws):
    return pl.pallas_call(
        _head_body,
        out_shape=jax.ShapeDtypeStruct((1, NC2), jnp.float32),
    )(pooled, pcnt, es, *ws)


def kernel(x, edge_patch, superpixel_attri, edge_superpixel, ln_in1_g,
           ln_in1_b, W1l, W1r, b1, ln_out1_g, ln_out1_b, ln_in2_g, ln_in2_b,
           W2l, W2r, b2, ln_out2_g, ln_out2_b, pe_W, pe_b, cls, pos, pre_g,
           pre_b, lnA_g, lnA_b, Wqkv, bqkv, Wo, bo, lnM_g, lnM_b, Wf1, bf1,
           Wf2, bf2, lnF_g, lnF_b, head_W, head_b):
    r = lambda v: v.reshape(1, -1)
    pad = N + jnp.arange(EPAD - E, dtype=jnp.int32) % (NP - N)
    src3d = jnp.concatenate([edge_patch[0], pad]).reshape(
        NW, NBODY * SGRP, SUB)
    dst3d = jnp.concatenate([edge_patch[1], pad]).reshape(
        NW, NBODY * SGRP, SUB)
    seg3d = superpixel_attri.reshape(GRID_N, 1, BN)
    x_pad = jnp.pad(x, ((0, NP - N), (0, 0)))

    h = _stage_ln(x_pad, r(ln_in1_g), r(ln_in1_b))
    part, cnt = _stage_sc(h, src3d, dst3d)
    cnt_col = _stage_cnt(cnt)
    pooled, pcnt = _stage_mid(h, part, cnt_col, seg3d, W1l, W1r, r(b1),
                              r(ln_out1_g), r(ln_out1_b))
    out = _stage_head(
        pooled, pcnt, edge_superpixel,
        r(ln_in2_g), r(ln_in2_b), W2l, W2r, r(b2), r(ln_out2_g), r(ln_out2_b),
        pe_W, r(pe_b), cls.reshape(1, D), pos.reshape(S + 1, D),
        r(pre_g), r(pre_b), r(lnA_g), r(lnA_b), Wqkv, r(bqkv), Wo, r(bo),
        r(lnM_g), r(lnM_b), Wf1, r(bf1), Wf2, r(bf2), r(lnF_g), r(lnF_b),
        head_W, r(head_b))
    return out
